# Initial kernel scaffold; baseline (speedup 1.0000x reference)
import jax
import jax.numpy as jnp
from jax.experimental import pallas as pl


def kernel(x, edge_index, batch, W_in, b_in, W_ih, W_hh, b_ih, b_hh, W_pred, b_pred):
    raise NotImplementedError("write your pallas kernel here")


# speedup vs baseline: 4.9496x; 4.9496x over previous
import jax
import jax.numpy as jnp
from jax import lax
from jax.experimental import pallas as pl
from jax.experimental.pallas import tpu as pltpu
from jax.experimental.pallas import tpu_sc as plsc

_N = 10000
_E = 320000
_D = 128
_G = 64
_NC = 2
_NS = 16
_NW = _NC * _NS
_EPT = _E // _NW
_CH = 80
_NCHUNK = _EPT // _CH
_NPAD = 10240
_RPT = _NPAD // _NS


def _sc_msg():
    mesh = plsc.VectorSubcoreMesh(core_axis_name="c", subcore_axis_name="s")
    out_type = jax.ShapeDtypeStruct((_NC, _NPAD, _D), jnp.float32)
    scratch = [
        pltpu.VMEM_SHARED((_NPAD, _D), jnp.float32),
        pltpu.VMEM((_CH,), jnp.int32),
        pltpu.VMEM((_CH,), jnp.int32),
        pltpu.VMEM((_CH, _D), jnp.float32),
        pltpu.SemaphoreType.DMA,
    ]

    def body(h_hbm, src_hbm, dst_hbm, z_hbm, msg_out,
             msg_sp, sidx, didx, rows, sem):
        c = lax.axis_index("c")
        s = lax.axis_index("s")
        wid = c * _NS + s
        pltpu.sync_copy(z_hbm, msg_sp.at[pl.ds(s * _RPT, _RPT)])
        plsc.subcore_barrier()
        base = wid * _EPT

        def chunk(ci, carry):
            off = base + ci * _CH
            pltpu.sync_copy(src_hbm.at[pl.ds(off, _CH)], sidx)
            pltpu.sync_copy(dst_hbm.at[pl.ds(off, _CH)], didx)
            pltpu.async_copy(h_hbm.at[sidx], rows, sem).wait()
            pltpu.sync_copy(rows, msg_sp.at[didx], add=True)
            return carry

        lax.fori_loop(0, _NCHUNK, chunk, 0)
        plsc.subcore_barrier()
        pltpu.sync_copy(msg_sp.at[pl.ds(s * _RPT, _RPT)],
                        msg_out.at[c, pl.ds(s * _RPT, _RPT)])

    return pl.kernel(body, out_type=out_type, mesh=mesh,
                     scratch_types=scratch)


def _sc_deg():
    mesh = plsc.VectorSubcoreMesh(core_axis_name="c", subcore_axis_name="s")
    out_type = jax.ShapeDtypeStruct((_NC, _NPAD, _D), jnp.float32)
    scratch = [
        pltpu.VMEM_SHARED((_NPAD, _D), jnp.float32),
        pltpu.VMEM((_CH,), jnp.int32),
        pltpu.VMEM((_CH, _D), jnp.float32),
    ]

    def body(dst_hbm, z_hbm, ones_hbm, deg_out, deg_sp, didx, ones_v):
        c = lax.axis_index("c")
        s = lax.axis_index("s")
        wid = c * _NS + s
        pltpu.sync_copy(z_hbm, deg_sp.at[pl.ds(s * _RPT, _RPT)])
        pltpu.sync_copy(ones_hbm, ones_v)
        plsc.subcore_barrier()
        base = wid * _EPT

        def chunk(ci, carry):
            off = base + ci * _CH
            pltpu.sync_copy(dst_hbm.at[pl.ds(off, _CH)], didx)
            pltpu.sync_copy(ones_v, deg_sp.at[didx], add=True)
            return carry

        lax.fori_loop(0, _NCHUNK, chunk, 0)
        plsc.subcore_barrier()
        pltpu.sync_copy(deg_sp.at[pl.ds(s * _RPT, _RPT)],
                        deg_out.at[c, pl.ds(s * _RPT, _RPT)])

    return pl.kernel(body, out_type=out_type, mesh=mesh,
                     scratch_types=scratch)


_SC_MSG = _sc_msg()
_SC_DEG = _sc_deg()


def _tc_input(x, w_t, b):
    def body(x_ref, w_ref, b_ref, o_ref):
        o_ref[...] = jnp.maximum(
            jnp.dot(x_ref[...], w_ref[...],
                    preferred_element_type=jnp.float32) + b_ref[...], 0.0)
    return pl.pallas_call(
        body, out_shape=jax.ShapeDtypeStruct((_N, _D), jnp.float32),
    )(x, w_t, b)


def _tc_update_first(h, m, deg):
    def body(h_ref, m_ref, deg_ref, o_ref, rd_ref):
        dsum = (deg_ref[0, pl.ds(0, _N), 0:1]
                + deg_ref[1, pl.ds(0, _N), 0:1])
        rd = 1.0 / jnp.maximum(dsum, 1.0)
        rd_ref[...] = rd
        o_ref[...] = (0.5 * h_ref[...]
                      + (0.5 * rd) * (m_ref[0, pl.ds(0, _N), :]
                                      + m_ref[1, pl.ds(0, _N), :]))
    return pl.pallas_call(
        body, out_shape=[jax.ShapeDtypeStruct((_N, _D), jnp.float32),
                         jax.ShapeDtypeStruct((_N, 1), jnp.float32)],
    )(h, m, deg)


def _tc_update(h, m, rd):
    def body(h_ref, m_ref, rd_ref, o_ref):
        o_ref[...] = (0.5 * h_ref[...]
                      + (0.5 * rd_ref[...]) * (m_ref[0, pl.ds(0, _N), :]
                                               + m_ref[1, pl.ds(0, _N), :]))
    return pl.pallas_call(
        body, out_shape=jax.ShapeDtypeStruct((_N, _D), jnp.float32),
    )(h, m, rd)


def _tc_s2s(h2, m, rd, batch_col, wih_t, whh_t, b_g, wpred_t, bp):
    def body(h_ref, m_ref, rd_ref, bc_ref, wih_ref, whh_ref, bg_ref,
             wp_ref, bp_ref, o_ref):
        h = (0.5 * h_ref[...]
             + (0.5 * rd_ref[...]) * (m_ref[0, pl.ds(0, _N), :]
                                      + m_ref[1, pl.ds(0, _N), :]))
        onehot = (bc_ref[...] ==
                  lax.broadcasted_iota(jnp.int32, (_N, _G), 1)
                  ).astype(jnp.float32)
        q_star = jnp.zeros((_G, 2 * _D), jnp.float32)
        hs = jnp.zeros((_G, _D), jnp.float32)
        cs = jnp.zeros((_G, _D), jnp.float32)
        for _ in range(3):
            gates = (jnp.dot(q_star, wih_ref[...],
                             preferred_element_type=jnp.float32)
                     + jnp.dot(hs, whh_ref[...],
                               preferred_element_type=jnp.float32)
                     + bg_ref[...])
            i_g = jax.nn.sigmoid(gates[:, :_D])
            f_g = jax.nn.sigmoid(gates[:, _D:2 * _D])
            g_g = jnp.tanh(gates[:, 2 * _D:3 * _D])
            o_g = jax.nn.sigmoid(gates[:, 3 * _D:])
            cs = f_g * cs + i_g * g_g
            hs = o_g * jnp.tanh(cs)
            q = hs
            scores = lax.dot_general(h, q, (((1,), (1,)), ((), ())),
                                     preferred_element_type=jnp.float32)
            masked = jnp.where(onehot > 0, scores, -1e30)
            seg_max = jnp.max(masked, axis=0, keepdims=True)
            e_max = jnp.max(jnp.where(onehot > 0, seg_max, -1e30),
                            axis=1, keepdims=True)
            e = jnp.sum(onehot * scores, axis=1, keepdims=True)
            a = jnp.exp(e - e_max)
            r_num = lax.dot_general(onehot, a * h,
                                    (((0,), (0,)), ((), ())),
                                    preferred_element_type=jnp.float32)
            a_sum = lax.dot_general(onehot, a,
                                    (((0,), (0,)), ((), ())),
                                    preferred_element_type=jnp.float32)
            r = r_num / jnp.maximum(a_sum, 1e-30)
            q_star = jnp.concatenate([q, r], axis=1)
        o_ref[...] = (jnp.dot(q_star, wp_ref[...],
                              preferred_element_type=jnp.float32)
                      + bp_ref[...])
    return pl.pallas_call(
        body, out_shape=jax.ShapeDtypeStruct((_G, 1), jnp.float32),
    )(h2, m, rd, batch_col, wih_t, whh_t, b_g, wpred_t, bp)


def kernel(x, edge_index, batch, W_in, b_in, W_ih, W_hh, b_ih, b_hh,
           W_pred, b_pred):
    x = x.astype(jnp.float32)
    src = edge_index[0].astype(jnp.int32)
    dst = edge_index[1].astype(jnp.int32)
    z = jnp.zeros((_RPT, _D), jnp.float32)
    ones = jnp.ones((_CH, _D), jnp.float32)

    deg = _SC_DEG(dst, z, ones)
    h0 = _tc_input(x, W_in.T, b_in.reshape(1, _D))
    msg = _SC_MSG(h0, src, dst, z)
    h1, rd = _tc_update_first(h0, msg, deg)
    msg = _SC_MSG(h1, src, dst, z)
    h2 = _tc_update(h1, msg, rd)
    msg = _SC_MSG(h2, src, dst, z)
    return _tc_s2s(h2, msg, rd,
                   batch.reshape(_N, 1).astype(jnp.int32),
                   W_ih.T, W_hh.T, (b_ih + b_hh).reshape(1, 4 * _D),
                   W_pred.T, b_pred.reshape(1, 1))
